# baseline (speedup 1.0000x reference)
import functools
import math

import jax
import jax.numpy as jnp
from jax import lax
from jax.experimental import pallas as pl
from jax.experimental.pallas import tpu as pltpu
from jax.experimental.pallas import tpu_sc as plsc

BATCH = 4096
SEQ = 200
DIM = 64
LANES = 16
NUM_CORES = 2
NUM_SUBCORES = 16
NW = NUM_CORES * NUM_SUBCORES
BLK = BATCH // NW
TOKW = BLK * SEQ
IROWS = TOKW // 128
HALF = BLK // 2
JG = LANES
SCALE = math.sqrt(DIM)


def _body(tok_hbm, lex2_hbm, pos2_hbm, out_hbm,
          raw_v, idx_v, par_v, pos_v, gbuf0, gbuf1, obuf0, obuf1,
          gsem0, gsem1, osem0, osem1):
    wid = lax.axis_index("s") * NUM_CORES + lax.axis_index("c")
    b0 = wid * BLK
    lanes = lax.iota(jnp.int32, LANES)

    pltpu.sync_copy(tok_hbm.at[pl.ds(wid * IROWS, IROWS)], raw_v)
    pltpu.sync_copy(pos2_hbm.at[pl.ds(0, 104)], pos_v)

    @plsc.parallel_loop(0, SEQ, unroll=2)
    def transpose(s):
        for jg in range(BLK // LANES):
            flat = jg * (LANES * SEQ) + lanes * SEQ + s
            raw = plsc.load_gather(raw_v, [flat >> 7, flat & 127])
            sl = pl.ds(jg * LANES, LANES)
            idx_v[s, sl] = raw >> 1
            par_v[s, sl] = (raw & 1) * DIM

    gbufs = (gbuf0, gbuf1)
    obufs = (obuf0, obuf1)
    gsems = (gsem0, gsem1)
    osems = (osem0, osem1)

    def fire(s, hb, gb):
        pltpu.make_async_copy(
            lex2_hbm.at[idx_v.at[s, pl.ds(hb * HALF, HALF)]],
            gbufs[gb], gsems[gb]).start()

    def wait_gather(gb):
        pltpu.make_async_copy(
            lex2_hbm.at[idx_v.at[0, pl.ds(0, HALF)]],
            gbufs[gb], gsems[gb]).wait()

    def start_out(s, ob):
        pltpu.make_async_copy(
            obufs[ob], out_hbm.at[s, :, pl.ds(b0, BLK)], osems[ob]).start()

    def wait_out(ob):
        pltpu.make_async_copy(
            obufs[ob], out_hbm.at[0, :, pl.ds(b0, BLK)], osems[ob]).wait()

    def compute(s, q, sb, hb, gb, ob):
        gbuf = gbufs[gb]
        obuf = obufs[ob]

        @plsc.parallel_loop(0, HALF // LANES, unroll=2)
        def jgroup(jg):
            rows = jg * LANES + lanes
            colbase = par_v[s, pl.ds(hb * HALF + jg * LANES, LANES)]
            for cg in range(DIM // LANES):
                pvec = pos_v[q, pl.ds(sb * DIM + cg * LANES, LANES)]
                for cl in range(LANES):
                    c = cg * LANES + cl
                    val = plsc.load_gather(gbuf, [rows, colbase + c])
                    obuf[c, pl.ds(hb * HALF + jg * LANES, LANES)] = (
                        val * SCALE + pvec[cl])
        return None

    fire(0, 0, 0)

    def body(q, _):
        for sb in range(2):
            s = 2 * q + sb
            for hb in range(2):
                h = 2 * s + hb
                nxt = h + 1

                @pl.when(nxt < 2 * SEQ)
                def _():
                    fire((nxt >> 1), nxt & 1, 1 - hb)

                if hb == 0:
                    @pl.when(q >= 1)
                    def _():
                        wait_out(sb)

                wait_gather(hb)
                compute(s, q, sb, hb, hb, sb)
                if hb == 1:
                    start_out(s, sb)
        return _

    lax.fori_loop(0, SEQ // 2, body, None)
    wait_out(0)
    wait_out(1)


@jax.jit
def _sc_embed(tok2, lex2, pos2):
    mesh = plsc.VectorSubcoreMesh(core_axis_name="c", subcore_axis_name="s")
    kern = functools.partial(
        pl.kernel,
        out_type=jax.ShapeDtypeStruct((SEQ, DIM, BATCH), jnp.float32),
        mesh=mesh,
        compiler_params=pltpu.CompilerParams(
            use_tc_tiling_on_sc=True, needs_layout_passes=False),
        scratch_types=[
            pltpu.VMEM((IROWS, 128), jnp.int32),
            pltpu.VMEM((SEQ, BLK), jnp.int32),
            pltpu.VMEM((SEQ, BLK), jnp.int32),
            pltpu.VMEM((104, 128), jnp.float32),
            pltpu.VMEM((HALF, 128), jnp.float32),
            pltpu.VMEM((HALF, 128), jnp.float32),
            pltpu.VMEM((DIM, BLK), jnp.float32),
            pltpu.VMEM((DIM, BLK), jnp.float32),
            pltpu.SemaphoreType.DMA,
            pltpu.SemaphoreType.DMA,
            pltpu.SemaphoreType.DMA,
            pltpu.SemaphoreType.DMA,
        ],
    )(_body)
    return kern(tok2, lex2, pos2)


def kernel(token_indices, lexical_weight, positional_weight):
    b, s = token_indices.shape
    v, d = lexical_weight.shape
    m, _ = positional_weight.shape
    tok2 = token_indices.reshape(b * s // 128, 128)
    lex2 = lexical_weight.reshape(v // 2, 2 * d)
    pos2 = positional_weight.reshape(m * d // 128, 128)
    out = _sc_embed(tok2, lex2, pos2)
    return jnp.transpose(out, (2, 0, 1))
